# baseline (speedup 1.0000x reference)
import functools

import jax
import jax.numpy as jnp
from jax import lax
from jax.experimental import pallas as pl
from jax.experimental.pallas import tpu as pltpu
from jax.experimental.pallas import tpu_sc as plsc

BATCH = 4096
SEQ = 200
DIM = 64
NUM_WORKERS = 32
CHUNK = 128
NB = BATCH // CHUNK
N_PAIRS = SEQ // 2


def _make_gather():
    mesh = plsc.VectorSubcoreMesh(core_axis_name="c", subcore_axis_name="s")

    @functools.partial(
        pl.kernel,
        mesh=mesh,
        out_type=jax.ShapeDtypeStruct((SEQ, 8, NB, 8, CHUNK), jnp.float32),
        scratch_types=[
            pltpu.VMEM((SEQ, CHUNK), jnp.int32),
            pltpu.VMEM((CHUNK, DIM), jnp.float32),
            pltpu.VMEM((CHUNK, DIM), jnp.float32),
            pltpu.VMEM((8, 8, CHUNK), jnp.float32),
            pltpu.VMEM((8, 8, CHUNK), jnp.float32),
            pltpu.SemaphoreType.DMA,
            pltpu.SemaphoreType.DMA,
            pltpu.SemaphoreType.DMA,
            pltpu.SemaphoreType.DMA,
        ],
        compiler_params=pltpu.CompilerParams(use_tc_tiling_on_sc=False,
                                             needs_layout_passes=False),
    )
    def gather_kernel(idx_hbm, table_hbm, out_hbm, idx_v, r_a, r_b, t_a, t_b,
                      g_a, g_b, o_a, o_b):
        w = lax.axis_index("s") * 2 + lax.axis_index("c")
        pltpu.sync_copy(idx_hbm.at[:, w], idx_v)

        lane = lax.iota(jnp.int32, 16)
        c_idx = [lane + 16 * gi for gi in range(8)]

        def fire(s, rbuf, sem):
            pltpu.async_copy(table_hbm.at[idx_v.at[s]], rbuf, sem)

        def gather_wait(rbuf, sem):
            pltpu.make_async_copy(table_hbm.at[idx_v.at[0]], rbuf, sem).wait()

        def transpose(rbuf, tbuf):
            def te_body(te, carry):
                for r in range(8):
                    e_idx = jnp.zeros((16,), jnp.int32) + (8 * te + r)
                    for gi in range(8):
                        tbuf[te, r, pl.ds(16 * gi, 16)] = plsc.load_gather(
                            rbuf, [c_idx[gi], e_idx])
                return carry
            lax.fori_loop(0, 8, te_body, 0)

        def out_start(s, tbuf, sem):
            pltpu.async_copy(tbuf, out_hbm.at[s, :, w], sem)

        def out_wait(tbuf, sem):
            pltpu.make_async_copy(tbuf, out_hbm.at[0, :, w], sem).wait()

        def half(s_proc, s_fire, rbuf, tbuf, gsem, osem, rother, first):
            gather_wait(rbuf, gsem)
            if not first:
                out_wait(tbuf, osem)
            transpose(rbuf, tbuf)
            out_start(s_proc, tbuf, osem)
            fire(s_fire, rbuf, gsem)

        fire(0, r_a, g_a)
        fire(1, r_b, g_b)
        half(0, 2, r_a, t_a, g_a, o_a, r_b, True)
        half(1, 3, r_b, t_b, g_b, o_b, r_a, True)

        def body(p, carry):
            s0 = 2 * p
            s1 = s0 + 1
            half(s0, jnp.minimum(s0 + 2, SEQ - 1), r_a, t_a, g_a, o_a, r_b,
                 False)
            half(s1, jnp.minimum(s1 + 2, SEQ - 1), r_b, t_b, g_b, o_b, r_a,
                 False)
            return carry

        lax.fori_loop(1, N_PAIRS, body, 0)
        gather_wait(r_a, g_a)
        gather_wait(r_b, g_b)
        out_wait(t_a, o_a)
        out_wait(t_b, o_b)

    return gather_kernel


_gather = _make_gather()


def kernel(word_inputs, word_seq_lengths, char_inputs, char_seq_lengths,
           char_seq_recover, word_embeddings):
    idx = word_inputs.T.astype(jnp.int32).reshape(SEQ, NB, CHUNK)
    x = _gather(idx, word_embeddings)
    return x.transpose(2, 4, 0, 1, 3).reshape(BATCH, SEQ, DIM)


# speedup vs baseline: 1.5446x; 1.5446x over previous
import functools

import jax
import jax.numpy as jnp
from jax import lax
from jax.experimental import pallas as pl
from jax.experimental.pallas import tpu as pltpu
from jax.experimental.pallas import tpu_sc as plsc

BATCH = 4096
SEQ = 200
DIM = 64
PAD = 128
NUM_WORKERS = 32
CHUNK = 128
NB = BATCH // CHUNK
N_PAIRS = SEQ // 2


def _make_gather():
    mesh = plsc.VectorSubcoreMesh(core_axis_name="c", subcore_axis_name="s")

    @functools.partial(
        pl.kernel,
        mesh=mesh,
        out_type=jax.ShapeDtypeStruct((SEQ, 8, NB, 8, CHUNK), jnp.float32),
        scratch_types=[
            pltpu.VMEM((SEQ, CHUNK), jnp.int32),
            pltpu.VMEM((CHUNK, PAD), jnp.float32),
            pltpu.VMEM((CHUNK, PAD), jnp.float32),
            pltpu.VMEM((8, 8, CHUNK), jnp.float32),
            pltpu.VMEM((8, 8, CHUNK), jnp.float32),
            pltpu.SemaphoreType.DMA,
            pltpu.SemaphoreType.DMA,
            pltpu.SemaphoreType.DMA,
            pltpu.SemaphoreType.DMA,
        ],
        compiler_params=pltpu.CompilerParams(needs_layout_passes=False),
    )
    def gather_kernel(idx_hbm, table_hbm, out_hbm, idx_v, r_a, r_b, t_a, t_b,
                      g_a, g_b, o_a, o_b):
        w = lax.axis_index("s") * 2 + lax.axis_index("c")
        pltpu.sync_copy(idx_hbm.at[:, w], idx_v)

        lane = lax.iota(jnp.int32, 16)
        c_idx = [lane + 16 * gi for gi in range(8)]

        def fire(s, rbuf, sem):
            pltpu.async_copy(table_hbm.at[idx_v.at[s]], rbuf, sem)

        def gather_wait(rbuf, sem):
            pltpu.make_async_copy(table_hbm.at[idx_v.at[0]], rbuf, sem).wait()

        def transpose(rbuf, tbuf):
            @plsc.parallel_loop(0, DIM, unroll=4)
            def e_body(e):
                te = lax.div(e, 8)
                r = lax.rem(e, 8)
                e_idx = jnp.zeros((16,), jnp.int32) + e
                for gi in range(8):
                    tbuf[te, r, pl.ds(16 * gi, 16)] = plsc.load_gather(
                        rbuf, [c_idx[gi], e_idx])

        def out_start(s, tbuf, sem):
            pltpu.async_copy(tbuf, out_hbm.at[s, :, w], sem)

        def out_wait(tbuf, sem):
            pltpu.make_async_copy(tbuf, out_hbm.at[0, :, w], sem).wait()

        def half(s_proc, s_fire, rbuf, tbuf, gsem, osem, first):
            gather_wait(rbuf, gsem)
            if not first:
                out_wait(tbuf, osem)
            transpose(rbuf, tbuf)
            out_start(s_proc, tbuf, osem)
            fire(s_fire, rbuf, gsem)

        fire(0, r_a, g_a)
        fire(1, r_b, g_b)
        half(0, 2, r_a, t_a, g_a, o_a, True)
        half(1, 3, r_b, t_b, g_b, o_b, True)

        def body(p, carry):
            s0 = 2 * p
            half(s0, jnp.minimum(s0 + 2, SEQ - 1), r_a, t_a, g_a, o_a, False)
            half(s0 + 1, jnp.minimum(s0 + 3, SEQ - 1), r_b, t_b, g_b, o_b,
                 False)
            return carry

        lax.fori_loop(1, N_PAIRS, body, 0)
        gather_wait(r_a, g_a)
        gather_wait(r_b, g_b)
        out_wait(t_a, o_a)
        out_wait(t_b, o_b)

    return gather_kernel


_gather = _make_gather()


def kernel(word_inputs, word_seq_lengths, char_inputs, char_seq_lengths,
           char_seq_recover, word_embeddings):
    idx = word_inputs.T.astype(jnp.int32).reshape(SEQ, NB, CHUNK)
    table = jnp.pad(word_embeddings, ((0, 0), (0, PAD - DIM)))
    x = _gather(idx, table)
    return x.transpose(2, 4, 0, 1, 3).reshape(BATCH, SEQ, DIM)
